# Initial kernel scaffold; baseline (speedup 1.0000x reference)
import jax
import jax.numpy as jnp
from jax.experimental import pallas as pl


def kernel(x, edge_index, edge_attr, batch, W_pre, b_pre, Wf, bf, Ws, bs, gamma, beta, W_post, b_post, W_out, b_out):
    raise NotImplementedError("write your pallas kernel here")


# speedup vs baseline: 3.3284x; 3.3284x over previous
import jax
import jax.numpy as jnp
from jax import lax
from jax.experimental import pallas as pl
from jax.experimental.pallas import tpu as pltpu
from jax.experimental.pallas import tpu_sc as plsc

N = 10000
E = 320000
D_IN = 128
D = 64
G = 128
W = 128

NC = 2
NS = 16
NW = NC * NS
EPT = E // NW
C = 80
CHUNKS = EPT // C
NPAD = 10240
NPT = NPAD // NS
ZROWS = 128

_mesh = plsc.VectorSubcoreMesh(core_axis_name="c", subcore_axis_name="s")


def _sc_layer_body(pd_hbm, ps_hbm, ee_hbm, dst_hbm, src_hbm, zeros_hbm,
                   out_hbm, dstbuf, srcbuf, sbuf, gd, gs, msgbuf, acc,
                   sem1, sem2):
    c = lax.axis_index("c")
    s = lax.axis_index("s")
    wid = s * NC + c

    zero16 = jnp.zeros((16,), jnp.float32)
    for j in range(NPT // ZROWS):
        pltpu.sync_copy(zeros_hbm, acc.at[pl.ds(s * NPT + j * ZROWS, ZROWS)])

    one_hot0 = jnp.where(jax.lax.iota(jnp.int32, 16) == 0, 1.0, 0.0)

    def fill_pad(e, _):
        msgbuf[e, pl.ds(64, 16)] = one_hot0
        for v in range(5, 8):
            msgbuf[e, pl.ds(v * 16, 16)] = zero16
        return 0

    lax.fori_loop(0, C, fill_pad, 0)
    plsc.subcore_barrier()

    def chunk(k, _):
        gbase = wid * EPT + k * C
        pltpu.sync_copy(dst_hbm.at[pl.ds(gbase, C)], dstbuf)
        pltpu.sync_copy(src_hbm.at[pl.ds(gbase, C)], srcbuf)
        cp1 = pltpu.async_copy(pd_hbm.at[dstbuf], gd, sem1)
        cp2 = pltpu.async_copy(ps_hbm.at[srcbuf], gs, sem2)
        pltpu.sync_copy(ee_hbm.at[pl.ds(gbase, C), :], sbuf)
        cp1.wait()
        cp2.wait()

        def edge(e, _):
            for v in range(4):
                af = (sbuf[e, pl.ds(v * 16, 16)]
                      + gd[e, pl.ds(v * 16, 16)]
                      + gs[e, pl.ds(v * 16, 16)])
                asv = (sbuf[e, pl.ds(64 + v * 16, 16)]
                       + gd[e, pl.ds(64 + v * 16, 16)]
                       + gs[e, pl.ds(64 + v * 16, 16)])
                sig = 1.0 / (1.0 + jnp.exp(-af))
                t = jnp.exp(-jnp.abs(asv))
                u = t / (2.0 + t)
                u2 = u * u
                poly = u * (1.0 + u2 * (1.0 / 3.0 + u2 * (0.2 + u2 * (
                    1.0 / 7.0 + u2 * (1.0 / 9.0)))))
                sp = jnp.maximum(asv, 0.0) + 2.0 * poly
                msgbuf[e, pl.ds(v * 16, 16)] = sig * sp
            return 0

        lax.fori_loop(0, C, edge, 0)
        pltpu.sync_copy(msgbuf, acc.at[dstbuf], add=True)
        return 0

    lax.fori_loop(0, CHUNKS, chunk, 0)
    plsc.subcore_barrier()
    pltpu.sync_copy(acc.at[pl.ds(s * NPT, NPT)],
                    out_hbm.at[c, pl.ds(s * NPT, NPT)])


_sc_layer = pl.kernel(
    _sc_layer_body,
    out_type=jax.ShapeDtypeStruct((NC, NPAD, W), jnp.float32),
    mesh=_mesh,
    scratch_types=[
        pltpu.VMEM((C,), jnp.int32),
        pltpu.VMEM((C,), jnp.int32),
        pltpu.VMEM((C, W), jnp.float32),
        pltpu.VMEM((C, W), jnp.float32),
        pltpu.VMEM((C, W), jnp.float32),
        pltpu.VMEM((C, W), jnp.float32),
        pltpu.VMEM_SHARED((NPAD, W), jnp.float32),
        pltpu.SemaphoreType.DMA,
        pltpu.SemaphoreType.DMA,
    ],
)


def _tc_pre_body(x_ref, wpre_ref, bpre_ref, wd_ref, ws_ref,
                 out_ref, pd_ref, psrc_ref):
    out = jnp.maximum(
        jnp.dot(x_ref[...], wpre_ref[...],
                preferred_element_type=jnp.float32) + bpre_ref[...], 0.0)
    out_ref[...] = out
    pd_ref[...] = jnp.dot(out, wd_ref[...], preferred_element_type=jnp.float32)
    psrc_ref[...] = jnp.dot(out, ws_ref[...],
                            preferred_element_type=jnp.float32)


_tc_pre = pl.pallas_call(
    _tc_pre_body,
    out_shape=(
        jax.ShapeDtypeStruct((N, D), jnp.float32),
        jax.ShapeDtypeStruct((N, 2 * D), jnp.float32),
        jax.ShapeDtypeStruct((N, 2 * D), jnp.float32),
    ),
)

_BE = 6400


def _tc_ee_body(eat_ref, w_ref, b_ref, e0_ref, e1_ref, e2_ref):
    ee = jax.lax.dot_general(
        eat_ref[...], w_ref[...], (((0,), (0,)), ((), ())),
        preferred_element_type=jnp.float32) + b_ref[...]
    e0_ref[...] = ee[:, 0:128]
    e1_ref[...] = ee[:, 128:256]
    e2_ref[...] = ee[:, 256:384]


_tc_ee = pl.pallas_call(
    _tc_ee_body,
    grid=(E // _BE,),
    in_specs=[
        pl.BlockSpec((3, _BE), lambda k: (0, k)),
        pl.BlockSpec((3, 384), lambda k: (0, 0)),
        pl.BlockSpec((1, 384), lambda k: (0, 0)),
    ],
    out_specs=(
        pl.BlockSpec((_BE, 128), lambda k: (k, 0)),
        pl.BlockSpec((_BE, 128), lambda k: (k, 0)),
        pl.BlockSpec((_BE, 128), lambda k: (k, 0)),
    ),
    out_shape=(
        jax.ShapeDtypeStruct((E, 128), jnp.float32),
        jax.ShapeDtypeStruct((E, 128), jnp.float32),
        jax.ShapeDtypeStruct((E, 128), jnp.float32),
    ),
)


def _bn_update(out_prev, p0, p1, gam, bet):
    deg = jnp.maximum(p0[:, D:D + 1] + p1[:, D:D + 1], 1.0)
    o = out_prev + (p0[:, :D] + p1[:, :D]) / deg
    mean = jnp.mean(o, axis=0, keepdims=True)
    var = jnp.mean((o - mean) * (o - mean), axis=0, keepdims=True)
    return (o - mean) * lax.rsqrt(var + 1e-5) * gam + bet


def _tc_mid_body(out_ref, p0_ref, p1_ref, gam_ref, bet_ref,
                 wd_ref, ws_ref, on_ref, pd_ref, psrc_ref):
    on = _bn_update(out_ref[...], p0_ref[...], p1_ref[...],
                    gam_ref[...], bet_ref[...])
    on_ref[...] = on
    pd_ref[...] = jnp.dot(on, wd_ref[...], preferred_element_type=jnp.float32)
    psrc_ref[...] = jnp.dot(on, ws_ref[...],
                            preferred_element_type=jnp.float32)


_tc_mid = pl.pallas_call(
    _tc_mid_body,
    out_shape=(
        jax.ShapeDtypeStruct((N, D), jnp.float32),
        jax.ShapeDtypeStruct((N, 2 * D), jnp.float32),
        jax.ShapeDtypeStruct((N, 2 * D), jnp.float32),
    ),
)


def _tc_final_body(out_ref, p0_ref, p1_ref, gam_ref, bet_ref,
                   batch_ref, wpost_ref, bpost_ref, wout_ref, bout_ref,
                   o_ref):
    on = _bn_update(out_ref[...], p0_ref[...], p1_ref[...],
                    gam_ref[...], bet_ref[...])
    gids = lax.broadcasted_iota(jnp.int32, (G, N), 0)
    oh = (batch_ref[...] == gids).astype(jnp.float32)
    cnt = jnp.maximum(jnp.sum(oh, axis=1, keepdims=True), 1.0)
    pooled = jnp.dot(oh, on, preferred_element_type=jnp.float32) / cnt
    h = jnp.maximum(
        jnp.dot(pooled, wpost_ref[...],
                preferred_element_type=jnp.float32) + bpost_ref[...], 0.0)
    o_ref[...] = jnp.dot(h, wout_ref[...],
                         preferred_element_type=jnp.float32) + bout_ref[...]


_tc_final = pl.pallas_call(
    _tc_final_body,
    out_shape=jax.ShapeDtypeStruct((G, 1), jnp.float32),
)


def kernel(x, edge_index, edge_attr, batch, W_pre, b_pre, Wf, bf, Ws, bs,
           gamma, beta, W_post, b_post, W_out, b_out):
    src = edge_index[0]
    dst = edge_index[1]

    wd = [jnp.concatenate([Wf[i][:D], Ws[i][:D]], axis=1) for i in range(3)]
    ws = [jnp.concatenate([Wf[i][D:2 * D], Ws[i][D:2 * D]], axis=1)
          for i in range(3)]
    wecat = jnp.concatenate(
        [jnp.concatenate([Wf[i][2 * D:], Ws[i][2 * D:]], axis=1)
         for i in range(3)], axis=1)
    becat = jnp.concatenate(
        [jnp.concatenate([bf[i], bs[i]]) for i in range(3)]).reshape(1, 384)

    ee = _tc_ee(edge_attr.T, wecat, becat)
    zrows = jnp.zeros((ZROWS, W), jnp.float32)
    out, pd, psrc = _tc_pre(x, W_pre, b_pre.reshape(1, D), wd[0], ws[0])

    for i in range(3):
        part = _sc_layer(pd, psrc, ee[i], dst, src, zrows)
        p0 = part[0, :N]
        p1 = part[1, :N]
        if i < 2:
            out, pd, psrc = _tc_mid(
                out, p0, p1, gamma[i].reshape(1, D),
                beta[i].reshape(1, D), wd[i + 1], ws[i + 1])
        else:
            o = _tc_final(
                out, p0, p1, gamma[i].reshape(1, D),
                beta[i].reshape(1, D),
                batch.reshape(1, N),
                W_post, b_post.reshape(1, D), W_out, b_out.reshape(1, 1))
    return o.reshape(-1)


# speedup vs baseline: 3.8696x; 1.1626x over previous
import jax
import jax.numpy as jnp
from jax import lax
from jax.experimental import pallas as pl
from jax.experimental.pallas import tpu as pltpu
from jax.experimental.pallas import tpu_sc as plsc

N = 10000
E = 320000
D_IN = 128
D = 64
G = 128
W = 128

NC = 2
NS = 16
NW = NC * NS
EPT = E // NW
C = 80
CHUNKS = EPT // C
NPAD = 10240
NPT = NPAD // NS
ZROWS = 128

_mesh = plsc.VectorSubcoreMesh(core_axis_name="c", subcore_axis_name="s")


def _sc_layer_body(pd_hbm, ps_hbm, ee_hbm, dst_hbm, src_hbm, zeros_hbm,
                   out_hbm, dstbuf, srcbuf, sbuf, gd, gs, msgbuf, acc,
                   sem1, sem2, isem1, isem2, esem):
    c = lax.axis_index("c")
    s = lax.axis_index("s")
    wid = s * NC + c

    zero16 = jnp.zeros((16,), jnp.float32)
    for j in range(NPT // ZROWS):
        pltpu.sync_copy(zeros_hbm, acc.at[pl.ds(s * NPT + j * ZROWS, ZROWS)])

    one_hot0 = jnp.where(jax.lax.iota(jnp.int32, 16) == 0, 1.0, 0.0)

    def fill_pad(e, _):
        msgbuf[e, pl.ds(64, 16)] = one_hot0
        for v in range(5, 8):
            msgbuf[e, pl.ds(v * 16, 16)] = zero16
        return 0

    lax.fori_loop(0, C, fill_pad, 0)
    plsc.subcore_barrier()

    def chunk(k, _):
        gbase = wid * EPT + k * C
        ci1 = pltpu.async_copy(dst_hbm.at[pl.ds(gbase, C)], dstbuf, isem1)
        ci2 = pltpu.async_copy(src_hbm.at[pl.ds(gbase, C)], srcbuf, isem2)
        ce = pltpu.async_copy(ee_hbm.at[pl.ds(gbase, C), :], sbuf, esem)
        ci1.wait()
        ci2.wait()
        cp1 = pltpu.async_copy(pd_hbm.at[dstbuf], gd, sem1)
        cp2 = pltpu.async_copy(ps_hbm.at[srcbuf], gs, sem2)
        ce.wait()
        cp1.wait()
        cp2.wait()

        def edge(e, _):
            for v in range(4):
                af = (sbuf[e, pl.ds(v * 16, 16)]
                      + gd[e, pl.ds(v * 16, 16)]
                      + gs[e, pl.ds(v * 16, 16)])
                asv = (sbuf[e, pl.ds(64 + v * 16, 16)]
                       + gd[e, pl.ds(64 + v * 16, 16)]
                       + gs[e, pl.ds(64 + v * 16, 16)])
                sig = 1.0 / (1.0 + jnp.exp(-af))
                t = jnp.exp(-jnp.abs(asv))
                u = t / (2.0 + t)
                u2 = u * u
                poly = u * (1.0 + u2 * (1.0 / 3.0 + u2 * (0.2 + u2 * (
                    1.0 / 7.0 + u2 * (1.0 / 9.0)))))
                sp = jnp.maximum(asv, 0.0) + 2.0 * poly
                msgbuf[e, pl.ds(v * 16, 16)] = sig * sp
            return 0

        lax.fori_loop(0, C, edge, 0)
        pltpu.sync_copy(msgbuf, acc.at[dstbuf], add=True)
        return 0

    lax.fori_loop(0, CHUNKS, chunk, 0)
    plsc.subcore_barrier()
    pltpu.sync_copy(acc.at[pl.ds(s * NPT, NPT)],
                    out_hbm.at[c, pl.ds(s * NPT, NPT)])


_sc_layer = pl.kernel(
    _sc_layer_body,
    out_type=jax.ShapeDtypeStruct((NC, NPAD, W), jnp.float32),
    mesh=_mesh,
    scratch_types=[
        pltpu.VMEM((C,), jnp.int32),
        pltpu.VMEM((C,), jnp.int32),
        pltpu.VMEM((C, W), jnp.float32),
        pltpu.VMEM((C, W), jnp.float32),
        pltpu.VMEM((C, W), jnp.float32),
        pltpu.VMEM((C, W), jnp.float32),
        pltpu.VMEM_SHARED((NPAD, W), jnp.float32),
        pltpu.SemaphoreType.DMA,
        pltpu.SemaphoreType.DMA,
        pltpu.SemaphoreType.DMA,
        pltpu.SemaphoreType.DMA,
        pltpu.SemaphoreType.DMA,
    ],
)


def _tc_pre_body(x_ref, wpre_ref, bpre_ref, wd_ref, ws_ref,
                 out_ref, pd_ref, psrc_ref):
    out = jnp.maximum(
        jnp.dot(x_ref[...], wpre_ref[...],
                preferred_element_type=jnp.float32) + bpre_ref[...], 0.0)
    out_ref[...] = out
    pd_ref[...] = jnp.dot(out, wd_ref[...], preferred_element_type=jnp.float32)
    psrc_ref[...] = jnp.dot(out, ws_ref[...],
                            preferred_element_type=jnp.float32)


_tc_pre = pl.pallas_call(
    _tc_pre_body,
    out_shape=(
        jax.ShapeDtypeStruct((N, D), jnp.float32),
        jax.ShapeDtypeStruct((N, 2 * D), jnp.float32),
        jax.ShapeDtypeStruct((N, 2 * D), jnp.float32),
    ),
)

_BE = 6400


def _tc_ee_body(eat_ref, w_ref, b_ref, e0_ref, e1_ref, e2_ref):
    ee = jax.lax.dot_general(
        eat_ref[...], w_ref[...], (((0,), (0,)), ((), ())),
        preferred_element_type=jnp.float32) + b_ref[...]
    e0_ref[...] = ee[:, 0:128]
    e1_ref[...] = ee[:, 128:256]
    e2_ref[...] = ee[:, 256:384]


_tc_ee = pl.pallas_call(
    _tc_ee_body,
    grid=(E // _BE,),
    in_specs=[
        pl.BlockSpec((3, _BE), lambda k: (0, k)),
        pl.BlockSpec((3, 384), lambda k: (0, 0)),
        pl.BlockSpec((1, 384), lambda k: (0, 0)),
    ],
    out_specs=(
        pl.BlockSpec((_BE, 128), lambda k: (k, 0)),
        pl.BlockSpec((_BE, 128), lambda k: (k, 0)),
        pl.BlockSpec((_BE, 128), lambda k: (k, 0)),
    ),
    out_shape=(
        jax.ShapeDtypeStruct((E, 128), jnp.float32),
        jax.ShapeDtypeStruct((E, 128), jnp.float32),
        jax.ShapeDtypeStruct((E, 128), jnp.float32),
    ),
)


def _bn_update(out_prev, p0, p1, gam, bet):
    deg = jnp.maximum(p0[:, D:D + 1] + p1[:, D:D + 1], 1.0)
    o = out_prev + (p0[:, :D] + p1[:, :D]) / deg
    mean = jnp.mean(o, axis=0, keepdims=True)
    var = jnp.mean((o - mean) * (o - mean), axis=0, keepdims=True)
    return (o - mean) * lax.rsqrt(var + 1e-5) * gam + bet


def _tc_mid_body(out_ref, p0_ref, p1_ref, gam_ref, bet_ref,
                 wd_ref, ws_ref, on_ref, pd_ref, psrc_ref):
    on = _bn_update(out_ref[...], p0_ref[...], p1_ref[...],
                    gam_ref[...], bet_ref[...])
    on_ref[...] = on
    pd_ref[...] = jnp.dot(on, wd_ref[...], preferred_element_type=jnp.float32)
    psrc_ref[...] = jnp.dot(on, ws_ref[...],
                            preferred_element_type=jnp.float32)


_tc_mid = pl.pallas_call(
    _tc_mid_body,
    out_shape=(
        jax.ShapeDtypeStruct((N, D), jnp.float32),
        jax.ShapeDtypeStruct((N, 2 * D), jnp.float32),
        jax.ShapeDtypeStruct((N, 2 * D), jnp.float32),
    ),
)


def _tc_final_body(out_ref, p0_ref, p1_ref, gam_ref, bet_ref,
                   batch_ref, wpost_ref, bpost_ref, wout_ref, bout_ref,
                   o_ref):
    on = _bn_update(out_ref[...], p0_ref[...], p1_ref[...],
                    gam_ref[...], bet_ref[...])
    gids = lax.broadcasted_iota(jnp.int32, (G, N), 0)
    oh = (batch_ref[...] == gids).astype(jnp.float32)
    cnt = jnp.maximum(jnp.sum(oh, axis=1, keepdims=True), 1.0)
    pooled = jnp.dot(oh, on, preferred_element_type=jnp.float32) / cnt
    h = jnp.maximum(
        jnp.dot(pooled, wpost_ref[...],
                preferred_element_type=jnp.float32) + bpost_ref[...], 0.0)
    o_ref[...] = jnp.dot(h, wout_ref[...],
                         preferred_element_type=jnp.float32) + bout_ref[...]


_tc_final = pl.pallas_call(
    _tc_final_body,
    out_shape=jax.ShapeDtypeStruct((G, 1), jnp.float32),
)


def kernel(x, edge_index, edge_attr, batch, W_pre, b_pre, Wf, bf, Ws, bs,
           gamma, beta, W_post, b_post, W_out, b_out):
    src = edge_index[0]
    dst = edge_index[1]

    wd = [jnp.concatenate([Wf[i][:D], Ws[i][:D]], axis=1) for i in range(3)]
    ws = [jnp.concatenate([Wf[i][D:2 * D], Ws[i][D:2 * D]], axis=1)
          for i in range(3)]
    wecat = jnp.concatenate(
        [jnp.concatenate([Wf[i][2 * D:], Ws[i][2 * D:]], axis=1)
         for i in range(3)], axis=1)
    becat = jnp.concatenate(
        [jnp.concatenate([bf[i], bs[i]]) for i in range(3)]).reshape(1, 384)

    ee = _tc_ee(edge_attr.T, wecat, becat)
    zrows = jnp.zeros((ZROWS, W), jnp.float32)
    out, pd, psrc = _tc_pre(x, W_pre, b_pre.reshape(1, D), wd[0], ws[0])

    for i in range(3):
        part = _sc_layer(pd, psrc, ee[i], dst, src, zrows)
        p0 = part[0, :N]
        p1 = part[1, :N]
        if i < 2:
            out, pd, psrc = _tc_mid(
                out, p0, p1, gamma[i].reshape(1, D),
                beta[i].reshape(1, D), wd[i + 1], ws[i + 1])
        else:
            o = _tc_final(
                out, p0, p1, gamma[i].reshape(1, D),
                beta[i].reshape(1, D),
                batch.reshape(1, N),
                W_post, b_post.reshape(1, D), W_out, b_out.reshape(1, 1))
    return o.reshape(-1)
